# Initial kernel scaffold; baseline (speedup 1.0000x reference)
import jax
import jax.numpy as jnp
from jax.experimental import pallas as pl


def kernel(node_feature, edge_index, edge_weight, node_states, W_xz, b_xz, W_hz, b_hz, W_xr, b_xr, W_hr, b_hr, W_xh, b_xh, W_hh, b_hh):
    raise NotImplementedError("write your pallas kernel here")


# speedup vs baseline: 1.0178x; 1.0178x over previous
import functools

import jax
import jax.numpy as jnp
from jax.experimental import pallas as pl
from jax.experimental.pallas import tpu as pltpu

N = 10000
E = 160000
D = 256
K = 7
BN = 1000


def _gates_kernel(x_ref, h_ref, tx_ref, th_ref, wx_ref, wh_ref, b_ref,
                  z_ref, hr_ref, cxh_ref):
    accx = jnp.dot(x_ref[...], wx_ref[0], preferred_element_type=jnp.float32)
    for k in range(K - 1):
        accx += jnp.dot(tx_ref[k], wx_ref[k + 1],
                        preferred_element_type=jnp.float32)
    acch = jnp.dot(h_ref[...], wh_ref[0], preferred_element_type=jnp.float32)
    for k in range(K - 1):
        acch += jnp.dot(th_ref[k], wh_ref[k + 1],
                        preferred_element_type=jnp.float32)
    z = jax.nn.sigmoid(accx[:, :D] + acch[:, :D] + b_ref[0][None, :])
    r = jax.nn.sigmoid(accx[:, D:2 * D] + acch[:, D:2 * D] + b_ref[1][None, :])
    z_ref[...] = z
    hr_ref[...] = h_ref[...] * r
    cxh_ref[...] = accx[:, 2 * D:] + b_ref[2][None, :]


def _gates_call(x, h, tx, th, wx, wh, b3):
    grid = (N // BN,)
    return pl.pallas_call(
        _gates_kernel,
        grid=grid,
        in_specs=[
            pl.BlockSpec((BN, D), lambda i: (i, 0)),
            pl.BlockSpec((BN, D), lambda i: (i, 0)),
            pl.BlockSpec((K - 1, BN, D), lambda i: (0, i, 0)),
            pl.BlockSpec((K - 1, BN, D), lambda i: (0, i, 0)),
            pl.BlockSpec((K, D, 3 * D), lambda i: (0, 0, 0)),
            pl.BlockSpec((K, D, 2 * D), lambda i: (0, 0, 0)),
            pl.BlockSpec((3, D), lambda i: (0, 0)),
        ],
        out_specs=[
            pl.BlockSpec((BN, D), lambda i: (i, 0)),
            pl.BlockSpec((BN, D), lambda i: (i, 0)),
            pl.BlockSpec((BN, D), lambda i: (i, 0)),
        ],
        out_shape=[
            jax.ShapeDtypeStruct((N, D), jnp.float32),
            jax.ShapeDtypeStruct((N, D), jnp.float32),
            jax.ShapeDtypeStruct((N, D), jnp.float32),
        ],
    )(x, h, tx, th, wx, wh, b3)


def _final_kernel(z_ref, h_ref, cxh_ref, hr_ref, thr_ref, whh_ref, b_ref,
                  out_ref):
    acc = jnp.dot(hr_ref[...], whh_ref[0], preferred_element_type=jnp.float32)
    for k in range(K - 1):
        acc += jnp.dot(thr_ref[k], whh_ref[k + 1],
                       preferred_element_type=jnp.float32)
    ht = jnp.tanh(cxh_ref[...] + acc + b_ref[0][None, :])
    z = z_ref[...]
    out_ref[...] = z * h_ref[...] + (1.0 - z) * ht


def _final_call(z, h, cxh, hr, thr, whh, bhh):
    grid = (N // BN,)
    return pl.pallas_call(
        _final_kernel,
        grid=grid,
        in_specs=[
            pl.BlockSpec((BN, D), lambda i: (i, 0)),
            pl.BlockSpec((BN, D), lambda i: (i, 0)),
            pl.BlockSpec((BN, D), lambda i: (i, 0)),
            pl.BlockSpec((BN, D), lambda i: (i, 0)),
            pl.BlockSpec((K - 1, BN, D), lambda i: (0, i, 0)),
            pl.BlockSpec((K, D, D), lambda i: (0, 0, 0)),
            pl.BlockSpec((1, D), lambda i: (0, 0)),
        ],
        out_specs=pl.BlockSpec((BN, D), lambda i: (i, 0)),
        out_shape=jax.ShapeDtypeStruct((N, D), jnp.float32),
    )(z, h, cxh, hr, thr, whh, bhh)


def _cheb_stack(v, src, dst, wn2):
    ts = []
    acc = jax.ops.segment_sum(wn2[:, None] * v[src], dst, num_segments=N)
    t_prev, t = v, 0.5 * acc
    ts.append(t)
    for _ in range(2, K):
        t_new = jax.ops.segment_sum(wn2[:, None] * t[src], dst,
                                    num_segments=N) - t_prev
        ts.append(t_new)
        t_prev, t = t, t_new
    return jnp.stack(ts)


def kernel(node_feature, edge_index, edge_weight, node_states,
           W_xz, b_xz, W_hz, b_hz, W_xr, b_xr, W_hr, b_hr,
           W_xh, b_xh, W_hh, b_hh):
    x = node_feature
    h = node_states
    src = edge_index[0]
    dst = edge_index[1]

    deg = jax.ops.segment_sum(edge_weight, src, num_segments=N)
    dinv = jnp.where(deg > 0, jax.lax.rsqrt(deg), 0.0)
    wn2 = -2.0 * dinv[src] * edge_weight * dinv[dst]

    tx = _cheb_stack(x, src, dst, wn2)
    th = _cheb_stack(h, src, dst, wn2)

    wx = jnp.concatenate([W_xz, W_xr, W_xh], axis=2)
    wh = jnp.concatenate([W_hz, W_hr], axis=2)
    b3 = jnp.stack([b_xz + b_hz, b_xr + b_hr, b_xh])

    z, hr, cxh = _gates_call(x, h, tx, th, wx, wh, b3)

    thr = _cheb_stack(hr, src, dst, wn2)
    return _final_call(z, h, cxh, hr, thr, W_hh, b_hh[None, :])


# speedup vs baseline: 2.2392x; 2.2001x over previous
import functools

import jax
import jax.numpy as jnp
from jax import lax
from jax.experimental import pallas as pl
from jax.experimental.pallas import tpu as pltpu
from jax.experimental.pallas import tpu_sc as plsc

N = 10000
E = 160000
D = 256
K = 7
BN = 400

NT = 16
CB = 128
NC = 80
EPT = NC * CB
EPAD = NT * EPT
NPAD = 10240
NPT = NPAD // NT
DR = 128
NDR = NPT // DR
DPT = NPAD // NT
HD = 64
NP = D // HD

_mesh = plsc.VectorSubcoreMesh(core_axis_name="c", subcore_axis_name="s")
_sc_params = pltpu.CompilerParams(needs_layout_passes=False,
                                  use_tc_tiling_on_sc=False)


def _vec_loop(ref2d, rows, fn):
    @pl.loop(0, rows)
    def _(r):
        for j in range(HD // 16):
            sl = (r, pl.ds(16 * j, 16))
            ref2d[sl] = fn(ref2d[sl])


def _wn_body(src_hbm, dst_hbm, w_hbm, out_hbm,
             srcv, dstv, wv, wnv, dinvv, degv, deg_sh, sem):
    s = lax.axis_index("s")

    pltpu.sync_copy(src_hbm.at[s], srcv)
    pltpu.sync_copy(dst_hbm.at[s], dstv)
    pltpu.sync_copy(w_hbm.at[s], wv)

    @pl.loop(0, DPT // 16)
    def _(g):
        degv[pl.ds(16 * g, 16)] = jnp.zeros((16,), jnp.float32)
    pltpu.sync_copy(degv, deg_sh.at[pl.ds(s * DPT, DPT)])
    plsc.subcore_barrier()

    @pl.loop(0, NC)
    def _(j):
        pltpu.async_copy(wv.at[j], deg_sh.at[srcv.at[j]], sem, add=True).wait()
    plsc.subcore_barrier()

    pltpu.sync_copy(deg_sh.at[pl.ds(s * DPT, DPT)], degv)

    @pl.loop(0, DPT // 16)
    def _(g):
        sl = pl.ds(16 * g, 16)
        x = degv[sl]
        sq = 0.5 * (1.0 + x)
        for _ in range(16):
            sq = 0.5 * (sq + x / sq)
        degv[sl] = jnp.where(x > 0.0, 1.0 / sq, 0.0)

    pltpu.sync_copy(degv, deg_sh.at[pl.ds(s * DPT, DPT)])
    plsc.subcore_barrier()

    pltpu.sync_copy(deg_sh, dinvv)

    @pl.loop(0, NC)
    def _(j):
        @pl.loop(0, CB // 16)
        def _(g):
            sl = (j, pl.ds(16 * g, 16))
            ds_ = plsc.load_gather(dinvv, [srcv[sl]])
            dd_ = plsc.load_gather(dinvv, [dstv[sl]])
            wnv[sl] = -2.0 * wv[sl] * ds_ * dd_
    pltpu.sync_copy(wnv, out_hbm.at[s])


def _wn_call(src_p, dst_p, w_p):
    return pl.kernel(
        _wn_body,
        out_type=jax.ShapeDtypeStruct((NT, NC, CB), jnp.float32),
        mesh=_mesh,
        scratch_types=[
            pltpu.VMEM((NC, CB), jnp.int32),
            pltpu.VMEM((NC, CB), jnp.int32),
            pltpu.VMEM((NC, CB), jnp.float32),
            pltpu.VMEM((NC, CB), jnp.float32),
            pltpu.VMEM((NPAD,), jnp.float32),
            pltpu.VMEM((DPT,), jnp.float32),
            pltpu.VMEM_SHARED((NPAD,), jnp.float32),
            pltpu.SemaphoreType.DMA,
        ],
        compiler_params=_sc_params,
    )(src_p, dst_p, w_p)


def _cheb_body(ppc, v_hbm, src_hbm, dst_hbm, wn_hbm, out_hbm,
               srcv, dstv, wnv, rows0, rows1, sb0, sb1, zbuf, da, db,
               acc_sh, gs0, gs1, ss0, ss1):
    c = lax.axis_index("c")
    s = lax.axis_index("s")
    rows = (rows0, rows1)
    sb = (sb0, sb1)
    gsem = (gs0, gs1)
    ssem = (ss0, ss1)

    pltpu.sync_copy(src_hbm.at[s], srcv.at[pl.ds(0, NC)])
    pltpu.sync_copy(dst_hbm.at[s], dstv.at[pl.ds(0, NC)])
    pltpu.sync_copy(wn_hbm.at[s], wnv)

    @pl.loop(0, CB // 16)
    def _(g):
        for extra in (NC, NC + 1):
            srcv[extra, pl.ds(16 * g, 16)] = jnp.zeros((16,), jnp.int32)

    _vec_loop(zbuf, DR, lambda v: jnp.zeros((16,), jnp.float32))

    for i in range(NDR):
        pltpu.sync_copy(zbuf, acc_sh.at[pl.ds(s * NPT + i * DR, DR)])
    plsc.subcore_barrier()

    @pl.loop(0, ppc)
    def _(pi):
        p = c * ppc + pi

        @pl.loop(0, NDR)
        def _(i):
            r0 = s * NPT + i * DR
            pltpu.sync_copy(v_hbm.at[p, pl.ds(r0, DR)], da)
            pltpu.sync_copy(da, out_hbm.at[0, p, pl.ds(r0, DR)])
        plsc.subcore_barrier()

        @pl.loop(1, K)
        def _(k):
            src_plane = out_hbm.at[k - 1, p]

            def issue_gather(j, b):
                return pltpu.async_copy(src_plane.at[srcv.at[j]], rows[b],
                                        gsem[b])

            def wait_gather(j, b):
                pltpu.make_async_copy(src_plane.at[srcv.at[j]], rows[b],
                                      gsem[b]).wait()

            def scale(j, b):
                @pl.loop(0, CB // 16)
                def _(g):
                    w16 = wnv[j, pl.ds(16 * g, 16)]
                    for l in range(16):
                        w = w16[l]
                        for jj in range(HD // 16):
                            sl = (16 * g + l, pl.ds(16 * jj, 16))
                            sb[b][sl] = w * rows[b][sl]

            def issue_scatter(j, b):
                return pltpu.async_copy(sb[b], acc_sh.at[dstv.at[j]],
                                        ssem[b], add=True)

            def wait_scatter(j, b):
                pltpu.make_async_copy(sb[b], acc_sh.at[dstv.at[j]],
                                      ssem[b]).wait()

            issue_gather(0, 0)
            issue_gather(1, 1)
            for b in range(2):
                wait_gather(b, b)
                scale(b, b)
                issue_gather(b + 2, b)
                issue_scatter(b, b)

            @pl.loop(2, NC, step=2)
            def _(j0):
                for b in range(2):
                    j = j0 + b
                    wait_gather(j, b)
                    wait_scatter(j - 2, b)
                    scale(j, b)
                    issue_gather(j + 2, b)
                    issue_scatter(j, b)

            for b in range(2):
                wait_scatter(NC - 2 + b, b)
                wait_gather(NC + b, b)
            plsc.subcore_barrier()

            fmul = jnp.where(k == 1, 0.5, 1.0)
            gmul = jnp.where(k == 1, 0.0, 1.0)
            km2 = jnp.maximum(k - 2, 0)

            @pl.loop(0, NDR)
            def _(i):
                r0 = s * NPT + i * DR
                pltpu.sync_copy(acc_sh.at[pl.ds(r0, DR)], da)
                pltpu.sync_copy(zbuf, acc_sh.at[pl.ds(r0, DR)])
                pltpu.sync_copy(out_hbm.at[km2, p, pl.ds(r0, DR)], db)

                @pl.loop(0, DR)
                def _(r):
                    for jj in range(HD // 16):
                        sl = (r, pl.ds(16 * jj, 16))
                        da[sl] = fmul * da[sl] - gmul * db[sl]
                pltpu.sync_copy(da, out_hbm.at[k, p, pl.ds(r0, DR)])
            plsc.subcore_barrier()


def _cheb_call(v_planes, src_p, dst_p, wn2_p):
    p_tot = v_planes.shape[0]
    body = functools.partial(_cheb_body, p_tot // 2)
    return pl.kernel(
        body,
        out_type=jax.ShapeDtypeStruct((K, p_tot, NPAD, HD), jnp.float32),
        mesh=_mesh,
        scratch_types=[
            pltpu.VMEM((NC + 2, CB), jnp.int32),
            pltpu.VMEM((NC, CB), jnp.int32),
            pltpu.VMEM((NC, CB), jnp.float32),
            pltpu.VMEM((CB, HD), jnp.float32),
            pltpu.VMEM((CB, HD), jnp.float32),
            pltpu.VMEM((CB, HD), jnp.float32),
            pltpu.VMEM((CB, HD), jnp.float32),
            pltpu.VMEM((DR, HD), jnp.float32),
            pltpu.VMEM((DR, HD), jnp.float32),
            pltpu.VMEM((DR, HD), jnp.float32),
            pltpu.VMEM_SHARED((NPAD, HD), jnp.float32),
            pltpu.SemaphoreType.DMA,
            pltpu.SemaphoreType.DMA,
            pltpu.SemaphoreType.DMA,
            pltpu.SemaphoreType.DMA,
        ],
        compiler_params=_sc_params,
    )(v_planes, src_p, dst_p, wn2_p)


def _gates_kernel(h_ref, t_ref, wxr_ref, whr_ref, b_ref,
                  z_ref, hr_ref, cxh_ref):
    accx = jnp.zeros((BN, 3 * D), jnp.float32)
    acch = jnp.zeros((BN, 2 * D), jnp.float32)
    for k in range(K):
        for p in range(NP):
            accx += jnp.dot(t_ref[k, p], wxr_ref[k, p],
                            preferred_element_type=jnp.float32)
            acch += jnp.dot(t_ref[k, NP + p], whr_ref[k, p],
                            preferred_element_type=jnp.float32)
    z = jax.nn.sigmoid(accx[:, :D] + acch[:, :D] + b_ref[0][None, :])
    r = jax.nn.sigmoid(accx[:, D:2 * D] + acch[:, D:2 * D] + b_ref[1][None, :])
    z_ref[...] = z
    hr_ref[...] = h_ref[...] * r
    cxh_ref[...] = accx[:, 2 * D:] + b_ref[2][None, :]


def _gates_call(h, t_xh, wxr, whr, b3):
    return pl.pallas_call(
        _gates_kernel,
        grid=(N // BN,),
        in_specs=[
            pl.BlockSpec((BN, D), lambda i: (i, 0)),
            pl.BlockSpec((K, 2 * NP, BN, HD), lambda i: (0, 0, i, 0)),
            pl.BlockSpec((K, NP, HD, 3 * D), lambda i: (0, 0, 0, 0)),
            pl.BlockSpec((K, NP, HD, 2 * D), lambda i: (0, 0, 0, 0)),
            pl.BlockSpec((3, D), lambda i: (0, 0)),
        ],
        out_specs=[
            pl.BlockSpec((BN, D), lambda i: (i, 0)),
            pl.BlockSpec((BN, D), lambda i: (i, 0)),
            pl.BlockSpec((BN, D), lambda i: (i, 0)),
        ],
        out_shape=[
            jax.ShapeDtypeStruct((N, D), jnp.float32),
            jax.ShapeDtypeStruct((N, D), jnp.float32),
            jax.ShapeDtypeStruct((N, D), jnp.float32),
        ],
    )(h, t_xh, wxr, whr, b3)


def _final_kernel(z_ref, h_ref, cxh_ref, thr_ref, whhr_ref, b_ref, out_ref):
    acc = jnp.zeros((BN, D), jnp.float32)
    for k in range(K):
        for p in range(NP):
            acc += jnp.dot(thr_ref[k, p], whhr_ref[k, p],
                           preferred_element_type=jnp.float32)
    ht = jnp.tanh(cxh_ref[...] + acc + b_ref[0][None, :])
    z = z_ref[...]
    out_ref[...] = z * h_ref[...] + (1.0 - z) * ht


def _final_call(z, h, cxh, thr, whhr, bhh):
    return pl.pallas_call(
        _final_kernel,
        grid=(N // BN,),
        in_specs=[
            pl.BlockSpec((BN, D), lambda i: (i, 0)),
            pl.BlockSpec((BN, D), lambda i: (i, 0)),
            pl.BlockSpec((BN, D), lambda i: (i, 0)),
            pl.BlockSpec((K, NP, BN, HD), lambda i: (0, 0, i, 0)),
            pl.BlockSpec((K, NP, HD, D), lambda i: (0, 0, 0, 0)),
            pl.BlockSpec((1, D), lambda i: (0, 0)),
        ],
        out_specs=pl.BlockSpec((BN, D), lambda i: (i, 0)),
        out_shape=jax.ShapeDtypeStruct((N, D), jnp.float32),
    )(z, h, cxh, thr, whhr, bhh)


def _planes(m):
    pl_ = m.reshape(N, NP, HD).transpose(1, 0, 2)
    return jnp.pad(pl_, ((0, 0), (0, NPAD - N), (0, 0)))


def kernel(node_feature, edge_index, edge_weight, node_states,
           W_xz, b_xz, W_hz, b_hz, W_xr, b_xr, W_hr, b_hr,
           W_xh, b_xh, W_hh, b_hh):
    x = node_feature
    h = node_states
    pad = EPAD - E
    src_p = jnp.concatenate(
        [edge_index[0], jnp.zeros((pad,), jnp.int32)]).reshape(NT, NC, CB)
    dst_p = jnp.concatenate(
        [edge_index[1], jnp.zeros((pad,), jnp.int32)]).reshape(NT, NC, CB)
    w_p = jnp.concatenate(
        [edge_weight, jnp.zeros((pad,), jnp.float32)]).reshape(NT, NC, CB)

    wn2 = _wn_call(src_p, dst_p, w_p)

    v_xh = jnp.concatenate([_planes(x), _planes(h)], axis=0)
    t_xh = _cheb_call(v_xh, src_p, dst_p, wn2)

    wx = jnp.concatenate([W_xz, W_xr, W_xh], axis=2)
    wh = jnp.concatenate([W_hz, W_hr], axis=2)
    b3 = jnp.stack([b_xz + b_hz, b_xr + b_hr, b_xh])

    z, hr, cxh = _gates_call(
        h, t_xh,
        wx.reshape(K, NP, HD, 3 * D),
        wh.reshape(K, NP, HD, 2 * D), b3)

    t_hr = _cheb_call(_planes(hr), src_p, dst_p, wn2)

    return _final_call(z, h, cxh, t_hr,
                       W_hh.reshape(K, NP, HD, D), b_hh[None, :])
